# Initial kernel scaffold; baseline (speedup 1.0000x reference)
import jax
import jax.numpy as jnp
from jax.experimental import pallas as pl


def kernel(feats):
    raise NotImplementedError("write your pallas kernel here")


# speedup vs baseline: 1.9983x; 1.9983x over previous
import functools

import jax
import jax.numpy as jnp
from jax import lax
from jax.experimental import pallas as pl
from jax.experimental.pallas import tpu as pltpu
from jax.experimental.pallas import tpu_sc as plsc

B = 8
T_IN = 4096
D = 512
CTX = 5
T_OUT = 1365
CH = 39
CPB = T_OUT // CH
NCHUNK = B * CPB
SLAB = 3 * CH + 2
NW = 32
MAXC = -(-NCHUNK // NW)

_mesh = plsc.VectorSubcoreMesh(core_axis_name="c", subcore_axis_name="s")


@functools.partial(
    pl.kernel,
    mesh=_mesh,
    out_type=jax.ShapeDtypeStruct((B * T_OUT * CTX * D,), jnp.float32),
    scratch_types=[
        pltpu.VMEM((SLAB * D,), jnp.float32),
        pltpu.SemaphoreType.DMA,
        pltpu.SemaphoreType.DMA,
    ],
)
def _splice(feats_hbm, out_hbm, slab, lsem, wsem):
    nc = 2
    wid = lax.axis_index("s") * nc + lax.axis_index("c")

    def chunk_body(i, carry):
        c = wid + i * NW

        @pl.when(c < NCHUNK)
        def _():
            b = c // CPB
            j = c - b * CPB
            t0 = j * CH
            ibase = b * (T_IN * D)
            obase = (b * T_OUT + t0) * (CTX * D)

            @pl.when(j == 0)
            def _():
                pltpu.async_copy(feats_hbm.at[pl.ds(ibase, (SLAB - 2) * D)],
                                 slab.at[pl.ds(2 * D, (SLAB - 2) * D)],
                                 lsem).wait()
                pltpu.async_copy(feats_hbm.at[pl.ds(ibase, D)],
                                 slab.at[pl.ds(0, D)], lsem).wait()
                pltpu.async_copy(feats_hbm.at[pl.ds(ibase, D)],
                                 slab.at[pl.ds(D, D)], lsem).wait()

            @pl.when(j != 0)
            def _():
                pltpu.async_copy(
                    feats_hbm.at[pl.ds(ibase + (3 * t0 - 2) * D, SLAB * D)],
                    slab, lsem).wait()

            copies = []
            for tp in range(CH):
                copies.append(pltpu.async_copy(
                    slab.at[pl.ds(3 * tp * D, CTX * D)],
                    out_hbm.at[pl.ds(obase + tp * (CTX * D), CTX * D)],
                    wsem))
            for cp in copies:
                cp.wait()

        return carry

    lax.fori_loop(0, MAXC, chunk_body, 0)


def kernel(feats):
    out = _splice(feats.reshape(-1))
    return out.reshape(B, T_OUT, CTX * D)
